# baseline (speedup 1.0000x reference)
import jax
import jax.numpy as jnp
from jax.experimental import pallas as pl
from jax.experimental.pallas import tpu as pltpu


def _gcn_body(adj1_ref, adj2_ref, seq_ref, w_ref, b_ref, alpha_ref, out_ref):
    wt = w_ref[...]
    b = b_ref[...]
    a = alpha_ref[0, 0]
    s = seq_ref[...]

    def halve(adj_ref):
        acc = jnp.dot(adj_ref[...], s, preferred_element_type=jnp.float32)
        o = jax.lax.dot_general(
            acc, wt, (((1,), (1,)), ((), ())),
            preferred_element_type=jnp.float32,
        )
        o = o + b
        return jnp.where(o >= 0.0, o, a * o)

    out_ref[0] = halve(adj1_ref)
    out_ref[1] = halve(adj2_ref)


def kernel(adj, seq, W, bias, alpha):
    B, M, K = adj.shape
    D = W.shape[0]
    adj2d = adj.reshape(M, K)
    seq2d = seq.reshape(K, seq.shape[2])
    bias2 = bias.reshape(1, D)
    alpha2 = jnp.asarray(alpha, jnp.float32).reshape(1, 1)

    MH = M // 2
    TM = 200
    nm = MH // TM

    out = pl.pallas_call(
        _gcn_body,
        grid=(nm,),
        in_specs=[
            pl.BlockSpec((TM, K), lambda m: (m, 0)),
            pl.BlockSpec((TM, K), lambda m, _nm=nm: (m + _nm, 0)),
            pl.BlockSpec((K, D), lambda m: (0, 0)),
            pl.BlockSpec((D, D), lambda m: (0, 0)),
            pl.BlockSpec((1, D), lambda m: (0, 0)),
            pl.BlockSpec((1, 1), lambda m: (0, 0)),
        ],
        out_specs=pl.BlockSpec((2, TM, D), lambda m: (0, m, 0)),
        out_shape=jax.ShapeDtypeStruct((2, MH, D), jnp.float32),
        compiler_params=pltpu.CompilerParams(
            dimension_semantics=("parallel",),
        ),
    )(adj2d, adj2d, seq2d, W, bias2, alpha2)
    return out.reshape(B, M, D)


# speedup vs baseline: 1.0719x; 1.0719x over previous
import jax
import jax.numpy as jnp
from jax.experimental import pallas as pl
from jax.experimental.pallas import tpu as pltpu


def _gcn_body(adj_ref, seq_ref, w_ref, b_ref, alpha_ref, out_ref):
    acc = jnp.dot(
        adj_ref[...], seq_ref[...], preferred_element_type=jnp.float32
    )
    o = jax.lax.dot_general(
        acc,
        w_ref[...],
        (((1,), (1,)), ((), ())),
        preferred_element_type=jnp.float32,
    )
    o = o + b_ref[...]
    out_ref[...] = jnp.where(o >= 0.0, o, alpha_ref[0, 0] * o)


def kernel(adj, seq, W, bias, alpha):
    B, M, K = adj.shape
    D = W.shape[0]
    adj2d = adj.reshape(M, K)
    seq2d = seq.reshape(K, seq.shape[2])
    bias2 = bias.reshape(1, D)
    alpha2 = jnp.asarray(alpha, jnp.float32).reshape(1, 1)

    TM = 512 if M > 512 else M
    grid = (pl.cdiv(M, TM),)

    out = pl.pallas_call(
        _gcn_body,
        grid=grid,
        in_specs=[
            pl.BlockSpec((TM, K), lambda m: (m, 0)),
            pl.BlockSpec((K, D), lambda m: (0, 0),
                         pipeline_mode=pl.Buffered(buffer_count=1)),
            pl.BlockSpec((D, D), lambda m: (0, 0)),
            pl.BlockSpec((1, D), lambda m: (0, 0)),
            pl.BlockSpec((1, 1), lambda m: (0, 0)),
        ],
        out_specs=pl.BlockSpec((TM, D), lambda m: (m, 0)),
        out_shape=jax.ShapeDtypeStruct((M, D), jnp.float32),
        compiler_params=pltpu.CompilerParams(
            dimension_semantics=("parallel",),
        ),
    )(adj2d, seq2d, W, bias2, alpha2)
    return out.reshape(B, M, D)


# speedup vs baseline: 1.0872x; 1.0143x over previous
import jax
import jax.numpy as jnp
from jax.experimental import pallas as pl
from jax.experimental.pallas import tpu as pltpu


def _gcn_body(adj_ref, seq_ref, w_ref, b_ref, alpha_ref, out_ref):
    acc = jnp.dot(
        adj_ref[...], seq_ref[...], preferred_element_type=jnp.float32
    )
    o = jax.lax.dot_general(
        acc,
        w_ref[...],
        (((1,), (1,)), ((), ())),
        preferred_element_type=jnp.float32,
    )
    o = o + b_ref[...]
    out_ref[...] = jnp.where(o >= 0.0, o, alpha_ref[0, 0] * o)


def kernel(adj, seq, W, bias, alpha):
    B, M, K = adj.shape
    D = W.shape[0]
    adj2d = adj.reshape(M, K)
    seq2d = seq.reshape(K, seq.shape[2])
    bias2 = bias.reshape(1, D)
    alpha2 = jnp.asarray(alpha, jnp.float32).reshape(1, 1)

    TM = 400 if M % 400 == 0 else M
    grid = (pl.cdiv(M, TM),)

    out = pl.pallas_call(
        _gcn_body,
        grid=grid,
        in_specs=[
            pl.BlockSpec((TM, K), lambda m: (m, 0)),
            pl.BlockSpec((K, D), lambda m: (0, 0),
                         pipeline_mode=pl.Buffered(buffer_count=1)),
            pl.BlockSpec((D, D), lambda m: (0, 0)),
            pl.BlockSpec((1, D), lambda m: (0, 0)),
            pl.BlockSpec((1, 1), lambda m: (0, 0)),
        ],
        out_specs=pl.BlockSpec((TM, D), lambda m: (m, 0)),
        out_shape=jax.ShapeDtypeStruct((M, D), jnp.float32),
        compiler_params=pltpu.CompilerParams(
            dimension_semantics=("parallel",),
        ),
    )(adj2d, seq2d, W, bias2, alpha2)
    return out.reshape(B, M, D)
